# baseline (speedup 1.0000x reference)
import functools

import jax
import jax.numpy as jnp
from jax import lax
from jax.experimental import pallas as pl
from jax.experimental.pallas import tpu as pltpu
from jax.experimental.pallas import tpu_sc as plsc

_D = 32
_NB = 16384
_S = 200
_NC = 2
_NS = 16
_NW = _NC * _NS
_BW = _NB // _NW
_CT = _BW // 128
_DT = _D // 8
_NBT = _NB // 128
_TR = _DT * _CT * 8
_PITCH = 129

_mesh = plsc.VectorSubcoreMesh(core_axis_name="c", subcore_axis_name="s")


@functools.partial(
    pl.kernel,
    mesh=_mesh,
    out_type=jax.ShapeDtypeStruct((_S * _DT * _NBT * 8, 128), jnp.float32),
    compiler_params=pltpu.CompilerParams(use_tc_tiling_on_sc=False,
                                         needs_layout_passes=False),
    scratch_types=(
        [pltpu.VMEM((_BW,), jnp.int32) for _ in range(2)]
        + [pltpu.VMEM((_BW, _D), jnp.float32) for _ in range(2)]
        + [pltpu.VMEM((_TR, _PITCH), jnp.float32) for _ in range(2)]
        + [pltpu.SemaphoreType.DMA for _ in range(6)]
    ),
)
def _embed_gather(idx_hbm, table_hbm, out_hbm, idx0, idx1, rows0, rows1,
                  tr0, tr1, isem0, isem1, gsem0, gsem1, osem0, osem1):
    idx_v = (idx0, idx1)
    rows_v = (rows0, rows1)
    tr_v = (tr0, tr1)
    isem = (isem0, isem1)
    gsem = (gsem0, gsem1)
    osem = (osem0, osem1)

    wid = lax.axis_index("s") * _NC + lax.axis_index("c")
    b0 = wid * _BW
    c0 = wid * _CT

    iota = lax.iota(jnp.int32, 16)
    r_lo = (lax.shift_right_logical(iota, 3) * (_CT * 8)
            + lax.bitwise_and(iota, 7))
    r_hi = r_lo + 2 * (_CT * 8)

    def idx_start(s, b):
        pltpu.async_copy(idx_hbm.at[pl.ds(s * _NB + b0, _BW)], idx_v[b],
                         isem[b])

    def idx_wait(b):
        pltpu.make_async_copy(idx_hbm.at[pl.ds(b0, _BW)], idx_v[b],
                              isem[b]).wait()

    def gather_start(b):
        pltpu.async_copy(table_hbm.at[idx_v[b]], rows_v[b], gsem[b])

    def gather_wait(b):
        pltpu.make_async_copy(table_hbm.at[idx_v[b]], rows_v[b],
                              gsem[b]).wait()

    def store_start(s, b):
        for t in range(_DT):
            pltpu.async_copy(
                tr_v[b].at[pl.ds(t * _CT * 8, _CT * 8), pl.ds(0, 128)],
                out_hbm.at[pl.ds((((s * _DT + t) * _NBT + c0) * 8),
                                 _CT * 8)],
                osem[b])

    def store_wait(b):
        for t in range(_DT):
            pltpu.make_async_copy(
                tr_v[b].at[pl.ds(t * _CT * 8, _CT * 8), pl.ds(0, 128)],
                out_hbm.at[pl.ds((t * _NBT + c0) * 8, _CT * 8)],
                osem[b]).wait()

    def transpose(b):
        rows = rows_v[b]
        tr = tr_v[b]

        @plsc.parallel_loop(0, _BW, unroll=8)
        def _(row):
            ct8 = lax.shift_right_logical(row, 7) * 8
            db = lax.bitwise_and(row, 127)
            lo = jnp.full((16,), ct8, jnp.int32) + r_lo
            hi = jnp.full((16,), ct8, jnp.int32) + r_hi
            dbs = jnp.full((16,), db, jnp.int32)
            plsc.store_scatter(tr, [lo, dbs], rows[row, pl.ds(0, 16)])
            plsc.store_scatter(tr, [hi, dbs], rows[row, pl.ds(16, 16)])

    def stage(s, b, do_idx, do_gather, first):
        nb = 1 - b
        gather_wait(b)
        if do_idx:
            idx_start(s + 2, b)
        if do_gather:
            idx_wait(nb)
            gather_start(nb)
        if not first:
            store_wait(b)
        transpose(b)
        store_start(s, b)

    idx_start(0, 0)
    idx_start(1, 1)
    idx_wait(0)
    gather_start(0)

    stage(0, 0, True, True, True)
    stage(1, 1, True, True, True)

    def body(jj, carry):
        s0 = 2 * jj
        stage(s0, 0, True, True, False)
        stage(s0 + 1, 1, True, True, False)
        return carry

    lax.fori_loop(1, _S // 2 - 1, body, 0)

    stage(_S - 2, 0, False, True, False)
    stage(_S - 1, 1, False, False, False)
    store_wait(0)
    store_wait(1)


def _repack_block(wt_ref, out_ref):
    x = wt_ref[...]
    eye = jnp.eye(32, dtype=jnp.float32)
    xt = lax.dot_general(x, eye, (((0,), (0,)), ((), ())),
                         precision=lax.Precision.HIGHEST,
                         preferred_element_type=jnp.float32)
    xg = xt.reshape(512, 4, 32)
    out_ref[...] = jnp.concatenate([xg[:, j, :] for j in range(4)], axis=1)


_repack = pl.pallas_call(
    _repack_block,
    grid=(489,),
    in_specs=[pl.BlockSpec((32, 2048), lambda i: (0, i))],
    out_specs=pl.BlockSpec((512, 128), lambda i: (i, 0)),
    out_shape=jax.ShapeDtypeStruct((250000, 128), jnp.float32),
)


def kernel(actions, weight):
    acts_t = actions.T.reshape(-1).astype(jnp.int32)
    table = _repack(weight.T).reshape(1000000, _D)
    out2 = _embed_gather(acts_t, table)
    return (out2.reshape(_S, _DT, _NBT, 8, 128)
            .transpose(2, 4, 0, 1, 3)
            .reshape(_NB, _S, _D))


# speedup vs baseline: 1.2919x; 1.2919x over previous
import functools

import jax
import jax.numpy as jnp
from jax import lax
from jax.experimental import pallas as pl
from jax.experimental.pallas import tpu as pltpu
from jax.experimental.pallas import tpu_sc as plsc

_D = 32
_NB = 16384
_S = 200
_NC = 2
_NS = 16
_NW = _NC * _NS
_BW = _NB // _NW
_CT = _BW // 128
_DT = _D // 8
_NBT = _NB // 128
_TR = _DT * _CT * 8
_PITCH = 129

_mesh = plsc.VectorSubcoreMesh(core_axis_name="c", subcore_axis_name="s")


@functools.partial(
    pl.kernel,
    mesh=_mesh,
    out_type=jax.ShapeDtypeStruct((_S * _DT * _NBT * 8, 128), jnp.float32),
    compiler_params=pltpu.CompilerParams(use_tc_tiling_on_sc=False,
                                         needs_layout_passes=False),
    scratch_types=(
        [pltpu.VMEM((_BW,), jnp.int32) for _ in range(2)]
        + [pltpu.VMEM((_BW, _D), jnp.float32) for _ in range(2)]
        + [pltpu.VMEM((_TR, _PITCH), jnp.float32) for _ in range(2)]
        + [pltpu.SemaphoreType.DMA for _ in range(6)]
    ),
)
def _embed_gather(idx_hbm, table_hbm, out_hbm, idx0, idx1, rows0, rows1,
                  tr0, tr1, isem0, isem1, gsem0, gsem1, osem0, osem1):
    idx_v = (idx0, idx1)
    rows_v = (rows0, rows1)
    tr_v = (tr0, tr1)
    isem = (isem0, isem1)
    gsem = (gsem0, gsem1)
    osem = (osem0, osem1)

    wid = lax.axis_index("s") * _NC + lax.axis_index("c")
    b0 = wid * _BW
    c0 = wid * _CT

    iota = lax.iota(jnp.int32, 16)
    r_lo = (lax.shift_right_logical(iota, 3) * (_CT * 8)
            + lax.bitwise_and(iota, 7))
    r_hi = r_lo + 2 * (_CT * 8)

    def idx_start(s, b):
        pltpu.async_copy(idx_hbm.at[pl.ds(s * _NB + b0, _BW)], idx_v[b],
                         isem[b])

    def idx_wait(b):
        pltpu.make_async_copy(idx_hbm.at[pl.ds(b0, _BW)], idx_v[b],
                              isem[b]).wait()

    def gather_start(b):
        pltpu.async_copy(table_hbm.at[idx_v[b]], rows_v[b], gsem[b])

    def gather_wait(b):
        pltpu.make_async_copy(table_hbm.at[idx_v[b]], rows_v[b],
                              gsem[b]).wait()

    def store_start(s, b):
        for t in range(_DT):
            pltpu.async_copy(
                tr_v[b].at[pl.ds(t * _CT * 8, _CT * 8), pl.ds(0, 128)],
                out_hbm.at[pl.ds((((s * _DT + t) * _NBT + c0) * 8),
                                 _CT * 8)],
                osem[b])

    def store_wait(b):
        for t in range(_DT):
            pltpu.make_async_copy(
                tr_v[b].at[pl.ds(t * _CT * 8, _CT * 8), pl.ds(0, 128)],
                out_hbm.at[pl.ds((t * _NBT + c0) * 8, _CT * 8)],
                osem[b]).wait()

    def transpose(b):
        rows = rows_v[b]
        tr = tr_v[b]

        @plsc.parallel_loop(0, _BW, unroll=8)
        def _(row):
            ct8 = lax.shift_right_logical(row, 7) * 8
            db = lax.bitwise_and(row, 127)
            lo = jnp.full((16,), ct8, jnp.int32) + r_lo
            hi = jnp.full((16,), ct8, jnp.int32) + r_hi
            dbs = jnp.full((16,), db, jnp.int32)
            plsc.store_scatter(tr, [lo, dbs], rows[row, pl.ds(0, 16)])
            plsc.store_scatter(tr, [hi, dbs], rows[row, pl.ds(16, 16)])

    def stage(s, b, do_idx, do_gather, first):
        nb = 1 - b
        gather_wait(b)
        if do_idx:
            idx_start(s + 2, b)
        if do_gather:
            idx_wait(nb)
            gather_start(nb)
        if not first:
            store_wait(b)
        transpose(b)
        store_start(s, b)

    idx_start(0, 0)
    idx_start(1, 1)
    idx_wait(0)
    gather_start(0)

    stage(0, 0, True, True, True)
    stage(1, 1, True, True, True)

    def body(jj, carry):
        s0 = 2 * jj
        stage(s0, 0, True, True, False)
        stage(s0 + 1, 1, True, True, False)
        return carry

    lax.fori_loop(1, _S // 2 - 1, body, 0)

    stage(_S - 2, 0, False, True, False)
    stage(_S - 1, 1, False, False, False)
    store_wait(0)
    store_wait(1)


def kernel(actions, weight):
    acts_t = actions.T.reshape(-1).astype(jnp.int32)
    out2 = _embed_gather(acts_t, weight)
    return (out2.reshape(_S, _DT, _NBT, 8, 128)
            .transpose(2, 4, 0, 1, 3)
            .reshape(_NB, _S, _D))
